# baseline (speedup 1.0000x reference)
import functools

import jax
import jax.numpy as jnp
from jax import lax
from jax.experimental import pallas as pl
from jax.experimental.pallas import tpu as pltpu
from jax.experimental.pallas import tpu_sc as plsc

_NC = 2
_NS = 16
_NW = _NC * _NS
_CHUNK = 128
_NBUF = 2
_L = 16


def _sc_gather(x2, idx3d):
    nw, chunks_per_w, chunk = idx3d.shape
    num_indices = nw * chunks_per_w * chunk
    b_per_w = chunks_per_w * chunk
    mesh = plsc.VectorSubcoreMesh(core_axis_name="c", subcore_axis_name="s")

    @functools.partial(
        pl.kernel,
        mesh=mesh,
        out_type=jax.ShapeDtypeStruct((num_indices // 2, 128), x2.dtype),
        scratch_types=[
            pltpu.VMEM((chunks_per_w, chunk), jnp.int32),
            pltpu.VMEM((chunks_per_w, chunk), jnp.int32),
            pltpu.VMEM((chunks_per_w, chunk), jnp.int32),
            *[pltpu.VMEM((chunk, 128), x2.dtype) for _ in range(_NBUF)],
            *[pltpu.VMEM((chunk // 2, 128), x2.dtype) for _ in range(_NBUF)],
            *[pltpu.SemaphoreType.DMA for _ in range(2 * _NBUF)],
        ],
    )
    def gather_kernel(table_hbm, idx_hbm, out_hbm, idx_v, phys_v, colb_v,
                      *rest):
        bufa = rest[:_NBUF]
        bufb = rest[_NBUF:2 * _NBUF]
        gsems = rest[2 * _NBUF:3 * _NBUF]
        wsems = rest[3 * _NBUF:]
        wid = lax.axis_index("s") * _NC + lax.axis_index("c")
        base = wid * b_per_w
        pltpu.sync_copy(idx_hbm.at[wid], idx_v)

        @pl.loop(0, chunks_per_w)
        def _(r):
            for g in range(chunk // _L):
                sl = pl.ds(g * _L, _L)
                v = idx_v[r, sl]
                phys_v[r, sl] = lax.shift_right_logical(v, 1)
                colb_v[r, sl] = (v & 1) * 64

        for b in range(_NBUF):
            pltpu.async_copy(table_hbm.at[phys_v.at[b]], bufa[b], gsems[b])

        @pl.loop(0, chunks_per_w, step=_NBUF)
        def _(j0):
            for b in range(_NBUF):
                j = j0 + b
                out_off = pl.multiple_of(
                    (base + j * chunk) // 2, chunk // 2)
                out_slice = out_hbm.at[pl.ds(out_off, chunk // 2)]
                pltpu.make_async_copy(
                    table_hbm.at[phys_v.at[j]], bufa[b], gsems[b]).wait()

                @pl.when(j >= _NBUF)
                def _():
                    pltpu.make_async_copy(bufb[b], out_slice, wsems[b]).wait()

                for g in range(chunk // _L):
                    pbv = colb_v[j, pl.ds(g * _L, _L)]
                    for l in range(_L):
                        row = g * _L + l
                        pb = pbv[l]
                        for k in range(64 // _L):
                            bufb[b][row // 2,
                                    pl.ds((row % 2) * 64 + k * _L, _L)] = (
                                bufa[b][row, pl.ds(pb + k * _L, _L)])

                pltpu.async_copy(bufb[b], out_slice, wsems[b])
                nxt = j + _NBUF

                @pl.when(nxt < chunks_per_w)
                def _():
                    pltpu.async_copy(
                        table_hbm.at[phys_v.at[nxt]], bufa[b], gsems[b])

        for b in range(_NBUF):
            pltpu.make_async_copy(
                bufb[b], out_hbm.at[pl.ds(0, chunk // 2)], wsems[b]).wait()

    return gather_kernel(x2, idx3d)


@jax.jit
def kernel(x, indices):
    b, s = indices.shape
    idx3d = indices.reshape(_NW, b * s // (_NW * _CHUNK), _CHUNK).astype(
        jnp.int32)
    x2 = x.reshape(x.shape[0] // 2, 128)
    out = _sc_gather(x2, idx3d)
    return out.reshape(b, s, x.shape[1])


# speedup vs baseline: 1.1099x; 1.1099x over previous
import functools

import jax
import jax.numpy as jnp
from jax import lax
from jax.experimental import pallas as pl
from jax.experimental.pallas import tpu as pltpu
from jax.experimental.pallas import tpu_sc as plsc

_NC = 2
_NS = 16
_NW = _NC * _NS
_CHUNK = 128
_NBUF = 5


def _sc_gather(x, idx2d):
    nchunks_total, chunk = idx2d.shape
    value_dim = x.shape[1]
    num_indices = nchunks_total * chunk
    chunks_per_w = nchunks_total // _NW
    b_per_w = chunks_per_w * chunk
    mesh = plsc.VectorSubcoreMesh(core_axis_name="c", subcore_axis_name="s")

    @functools.partial(
        pl.kernel,
        mesh=mesh,
        out_type=jax.ShapeDtypeStruct((num_indices, value_dim), x.dtype),
        scratch_types=[
            pltpu.VMEM((chunks_per_w, chunk), jnp.int32),
            *[pltpu.VMEM((chunk, value_dim), x.dtype) for _ in range(_NBUF)],
            *[pltpu.SemaphoreType.DMA for _ in range(2 * _NBUF)],
        ],
        compiler_params=pltpu.CompilerParams(use_tc_tiling_on_sc=False),
    )
    def gather_kernel(table_hbm, idx_hbm, out_hbm, idx_v, *rest):
        bufs = rest[:_NBUF]
        gsems = rest[_NBUF:2 * _NBUF]
        wsems = rest[2 * _NBUF:]
        wid = lax.axis_index("s") * _NC + lax.axis_index("c")
        base = wid * b_per_w
        pltpu.sync_copy(idx_hbm.at[pl.ds(wid * chunks_per_w, chunks_per_w)],
                        idx_v)

        for b in range(_NBUF):
            pltpu.async_copy(table_hbm.at[idx_v.at[b]], bufs[b], gsems[b])

        @pl.loop(0, chunks_per_w, step=_NBUF)
        def _(g):
            for b in range(_NBUF):
                c = g + b
                out_slice = out_hbm.at[pl.ds(base + c * chunk, chunk)]
                pltpu.make_async_copy(
                    table_hbm.at[idx_v.at[c]], bufs[b], gsems[b]).wait()
                pltpu.async_copy(bufs[b], out_slice, wsems[b])
                nxt = c + _NBUF

                @pl.when(nxt < chunks_per_w)
                def _():
                    pltpu.make_async_copy(bufs[b], out_slice, wsems[b]).wait()
                    pltpu.async_copy(
                        table_hbm.at[idx_v.at[nxt]], bufs[b], gsems[b])

        for b in range(_NBUF):
            pltpu.make_async_copy(
                bufs[b], out_hbm.at[pl.ds(base, chunk)], wsems[b]).wait()

    return gather_kernel(x, idx2d)


@jax.jit
def kernel(x, indices):
    b, s = indices.shape
    idx2d = indices.reshape(b * s // _CHUNK, _CHUNK).astype(jnp.int32)
    out = _sc_gather(x, idx2d)
    return out.reshape(b, s, x.shape[1])
